# Initial kernel scaffold; baseline (speedup 1.0000x reference)
import jax
import jax.numpy as jnp
from jax.experimental import pallas as pl


def kernel(x, knn_edge_index, ppi_edge_index, cWl0, cWr0, cb0, cWl1, cWr1, cb1, rWl0, rWr0, rb0, rWl1, rWr1, rb1, re_gW, re_gb, re_Wq, re_bq, re_Wk, re_bk, re_Wv, re_bv, re_Ws, re_bs, ce_gW, ce_gb, ce_Wq, ce_bq, ce_Wk, ce_bk, ce_Wv, ce_bv, ce_Ws, ce_bs, dW1, db1, dW2, db2, dW3, db3):
    raise NotImplementedError("write your pallas kernel here")


# speedup vs baseline: 8.0075x; 8.0075x over previous
import functools
import math

import jax
import jax.numpy as jnp
from jax.experimental import pallas as pl

_G, _C, _EPPI, _EKNN, _INTER, _EMB = 1024, 2048, 32768, 65536, 512, 128


def _leaky(v):
    return jnp.where(v > 0, v, 0.01 * v)


def _dot(a, b, ca, cb):
    return jax.lax.dot_general(
        a, b, (((ca,), (cb,)), ((), ())), preferred_element_type=jnp.float32
    )


def _cnt_body(src_ref, dst_ref, out_ref, *, n, ke):
    i = pl.program_id(0)

    @pl.when(i == 0)
    def _():
        out_ref[...] = jnp.zeros_like(out_ref)

    src = src_ref[...]
    dst = dst_ref[...]
    iota_s = jax.lax.broadcasted_iota(jnp.int32, (ke, n), 1)
    iota_d = jax.lax.broadcasted_iota(jnp.int32, (n, ke), 0)
    smask = (src == iota_s).astype(jnp.bfloat16)
    dmask = (dst == iota_d).astype(jnp.bfloat16)
    out_ref[...] += _dot(dmask, smask, 1, 0)


def _build_cnt(edge_index, n):
    e = edge_index.shape[1]
    ke = 1024
    src = edge_index[0].reshape(e, 1)
    dst = edge_index[1].reshape(1, e)
    return pl.pallas_call(
        functools.partial(_cnt_body, n=n, ke=ke),
        grid=(e // ke,),
        in_specs=[
            pl.BlockSpec((ke, 1), lambda i: (i, 0)),
            pl.BlockSpec((1, ke), lambda i: (0, i)),
        ],
        out_specs=pl.BlockSpec((n, n), lambda i: (0, 0)),
        out_shape=jax.ShapeDtypeStruct((n, n), jnp.float32),
    )(src, dst)


def _prep_body(cnt_ref, anorm_ref, agcn_ref, *, n, br):
    i = pl.program_id(0)
    cnt = cnt_ref[...]
    deg = jnp.sum(cnt, axis=1, keepdims=True)
    anorm_ref[...] = cnt / jnp.maximum(deg, 1.0)
    rows = jax.lax.broadcasted_iota(jnp.int32, (br, n), 0) + i * br
    cols = jax.lax.broadcasted_iota(jnp.int32, (br, n), 1)
    cnti = cnt + jnp.where(rows == cols, 1.0, 0.0)
    agcn_ref[...] = cnti


def _prep(cnt, n):
    br = 256
    anorm, cnti = pl.pallas_call(
        functools.partial(_prep_body, n=n, br=br),
        grid=(n // br,),
        in_specs=[pl.BlockSpec((br, n), lambda i: (i, 0))],
        out_specs=[
            pl.BlockSpec((br, n), lambda i: (i, 0)),
            pl.BlockSpec((br, n), lambda i: (i, 0)),
        ],
        out_shape=[
            jax.ShapeDtypeStruct((n, n), jnp.float32),
            jax.ShapeDtypeStruct((n, n), jnp.float32),
        ],
    )(cnt)

    def _gcn_norm_body(cnti_ref, dinv_ref, dcol_ref, out_ref):
        out_ref[...] = cnti_ref[...] * dinv_ref[...] * dcol_ref[...]

    def _dinv_body(cnti_ref, out_ref):
        deg = jnp.sum(cnti_ref[...], axis=1, keepdims=True)
        out_ref[...] = jnp.where(
            deg > 0, jax.lax.rsqrt(jnp.maximum(deg, 1e-12)), 0.0
        )

    dinv = pl.pallas_call(
        _dinv_body,
        grid=(n // br,),
        in_specs=[pl.BlockSpec((br, n), lambda i: (i, 0))],
        out_specs=pl.BlockSpec((br, 1), lambda i: (i, 0)),
        out_shape=jax.ShapeDtypeStruct((n, 1), jnp.float32),
    )(cnti)
    agcn = pl.pallas_call(
        _gcn_norm_body,
        grid=(n // br,),
        in_specs=[
            pl.BlockSpec((br, n), lambda i: (i, 0)),
            pl.BlockSpec((1, n), lambda i: (0, 0)),
            pl.BlockSpec((br, 1), lambda i: (i, 0)),
        ],
        out_specs=pl.BlockSpec((br, n), lambda i: (i, 0)),
        out_shape=jax.ShapeDtypeStruct((n, n), jnp.float32),
    )(cnti, dinv.reshape(1, n), dinv)
    return anorm, agcn


def _colsage_meanT_body(e_ref, anorm_ref, out_ref):
    out_ref[...] = _dot(e_ref[...], anorm_ref[...], 1, 1)


def _colsage_mm_body(wl_ref, wr_ref, b_ref, meanT_ref, e_ref, out_ref):
    z = _dot(wl_ref[...], meanT_ref[...], 0, 0)
    z += _dot(wr_ref[...], e_ref[...], 0, 0)
    z += b_ref[...].reshape(-1, 1)
    out_ref[...] = _leaky(z)


def _rowsage_mean_body(anorm_ref, e_ref, out_ref):
    out_ref[...] = _dot(anorm_ref[...], e_ref[...], 1, 0)


def _rowsage_mm_body(mean_ref, e_ref, wl_ref, wr_ref, b_ref, out_ref):
    z = _dot(mean_ref[...], wl_ref[...], 1, 0)
    z += _dot(e_ref[...], wr_ref[...], 1, 0)
    z += b_ref[...]
    out_ref[...] = _leaky(z)


def _gcn_row_body(e_ref, gw_ref, gb_ref, agcn_ref, out_ref):
    h = _dot(e_ref[...], gw_ref[...], 1, 0)
    out_ref[...] = _leaky(_dot(agcn_ref[...], h, 1, 0) + gb_ref[...])


def _gcn_col_body(e_ref, gw_ref, gb_ref, agcn_ref, out_ref):
    h = _dot(e_ref[...], gw_ref[...], 0, 0)
    out_ref[...] = _leaky(_dot(agcn_ref[...], h, 1, 0) + gb_ref[...])


def _tconv_softmax_body(h_ref, cnt_ref, wq_ref, bq_ref, wk_ref, bk_ref,
                        wv_ref, bv_ref, ws_ref, bs_ref, out_ref, *, cd):
    h = h_ref[...]
    q = _dot(h, wq_ref[...], 1, 0) + bq_ref[...]
    k = _dot(h, wk_ref[...], 1, 0) + bk_ref[...]
    v = _dot(h, wv_ref[...], 1, 0) + bv_ref[...]
    s = _dot(q, k, 1, 1) * (1.0 / math.sqrt(cd))
    cnt = cnt_ref[...]
    mask = cnt > 0
    smax = jnp.max(jnp.where(mask, s, -1e30), axis=1, keepdims=True)
    smax = jnp.where(smax > -1e29, smax, 0.0)
    ex = jnp.where(mask, jnp.exp(s - smax), 0.0)
    den = jnp.sum(cnt * ex, axis=1, keepdims=True)
    p = cnt * ex / (den + 1e-16)
    out_ref[...] = _dot(p, v, 1, 0) + _dot(h, ws_ref[...], 1, 0) + bs_ref[...]


def _tconv_sigmoid_body(h_ref, cnt_ref, wq_ref, bq_ref, wk_ref, bk_ref,
                        wv_ref, bv_ref, ws_ref, bs_ref, out_ref,
                        *, cd, ecount, scale):
    h = h_ref[...]
    q = _dot(h, wq_ref[...], 1, 0) + bq_ref[...]
    k = _dot(h, wk_ref[...], 1, 0) + bk_ref[...]
    v = _dot(h, wv_ref[...], 1, 0) + bv_ref[...]
    s = _dot(q, k, 1, 1) * (1.0 / math.sqrt(cd))
    cnt = cnt_ref[...]
    m = jnp.sum(cnt * s) / ecount
    var = jnp.sum(cnt * (s - m) ** 2) / (ecount - 1.0)
    z = (s - m) * (scale * jax.lax.rsqrt(var))
    sig = 1.0 / (1.0 + jnp.exp(-z))
    out_ref[...] = (
        _dot(cnt * sig, v, 1, 0) + _dot(h, ws_ref[...], 1, 0) + bs_ref[...]
    )


def _decoder_body(c_ref, w1_ref, b1_ref, w2_ref, b2_ref, w3_ref, b3_ref,
                  out_ref):
    h = jnp.maximum(_dot(c_ref[...], w1_ref[...], 1, 0) + b1_ref[...], 0.0)
    h = jnp.maximum(_dot(h, w2_ref[...], 1, 0) + b2_ref[...], 0.0)
    out_ref[...] = _dot(h, w3_ref[...], 1, 0) + b3_ref[...]


def _call(body, out_shape, *args, **static):
    return pl.pallas_call(
        functools.partial(body, **static) if static else body,
        out_shape=jax.ShapeDtypeStruct(out_shape, jnp.float32),
    )(*args)


def _rowsage_mm(mean, e, wl, wr, b):
    bn = 512
    return pl.pallas_call(
        _rowsage_mm_body,
        grid=(_C // bn,),
        in_specs=[
            pl.BlockSpec((_G, _C), lambda j: (0, 0)),
            pl.BlockSpec((_G, _C), lambda j: (0, 0)),
            pl.BlockSpec((_C, bn), lambda j: (0, j)),
            pl.BlockSpec((_C, bn), lambda j: (0, j)),
            pl.BlockSpec((1, bn), lambda j: (0, j)),
        ],
        out_specs=pl.BlockSpec((_G, bn), lambda j: (0, j)),
        out_shape=jax.ShapeDtypeStruct((_G, _C), jnp.float32),
    )(mean, e, wl, wr, b.reshape(1, _C))


def kernel(x, knn_edge_index, ppi_edge_index, cWl0, cWr0, cb0, cWl1, cWr1,
           cb1, rWl0, rWr0, rb0, rWl1, rWr1, rb1, re_gW, re_gb, re_Wq, re_bq,
           re_Wk, re_bk, re_Wv, re_bv, re_Ws, re_bs, ce_gW, ce_gb, ce_Wq,
           ce_bq, ce_Wk, ce_bk, ce_Wv, ce_bv, ce_Ws, ce_bs, dW1, db1, dW2,
           db2, dW3, db3):
    cnt_knn = _build_cnt(knn_edge_index, _C)
    cnt_ppi = _build_cnt(ppi_edge_index, _G)
    anorm_knn, agcn_knn = _prep(cnt_knn, _C)
    anorm_ppi, agcn_ppi = _prep(cnt_ppi, _G)

    e = x
    for cWl, cWr, cb, rWl, rWr, rb in (
        (cWl0, cWr0, cb0, rWl0, rWr0, rb0),
        (cWl1, cWr1, cb1, rWl1, rWr1, rb1),
    ):
        meanT = _call(_colsage_meanT_body, (_G, _C), e, anorm_knn)
        e = _call(_colsage_mm_body, (_G, _C), cWl, cWr, cb, meanT, e)
        mean = _call(_rowsage_mean_body, (_G, _C), anorm_ppi, e)
        e = _rowsage_mm(mean, e, rWl, rWr, rb)

    hr = _call(_gcn_row_body, (_G, _INTER), e, re_gW, re_gb.reshape(1, -1),
               agcn_ppi)
    rows_embd = _call(
        _tconv_softmax_body, (_G, _EMB), hr, cnt_ppi,
        re_Wq, re_bq.reshape(1, -1), re_Wk, re_bk.reshape(1, -1),
        re_Wv, re_bv.reshape(1, -1), re_Ws, re_bs.reshape(1, -1),
        cd=_EMB)

    hc = _call(_gcn_col_body, (_C, _INTER), e, ce_gW, ce_gb.reshape(1, -1),
               agcn_knn)
    cols_embd = _call(
        _tconv_sigmoid_body, (_C, _EMB), hc, cnt_knn,
        ce_Wq, ce_bq.reshape(1, -1), ce_Wk, ce_bk.reshape(1, -1),
        ce_Wv, ce_bv.reshape(1, -1), ce_Ws, ce_bs.reshape(1, -1),
        cd=_EMB, ecount=float(_EKNN), scale=3.0)

    out_features = _call(
        _decoder_body, (_C, _G), cols_embd, dW1, db1.reshape(1, -1),
        dW2, db2.reshape(1, -1), dW3, db3.reshape(1, -1))

    return (rows_embd, cols_embd, out_features)


# speedup vs baseline: 14.7843x; 1.8463x over previous
import functools
import math

import jax
import jax.numpy as jnp
from jax import lax
from jax.experimental import pallas as pl
from jax.experimental.pallas import tpu as pltpu
from jax.experimental.pallas import tpu_sc as plsc

_G, _C, _EPPI, _EKNN, _INTER, _EMB = 1024, 2048, 32768, 65536, 512, 128


def _leaky(v):
    return jnp.where(v > 0, v, 0.01 * v)


def _dot(a, b, ca, cb):
    return jax.lax.dot_general(
        a, b, (((ca,), (cb,)), ((), ())), preferred_element_type=jnp.float32
    )


def _cnt_body(src_ref, dst_ref, out_ref, *, n, ke):
    i = pl.program_id(0)

    @pl.when(i == 0)
    def _():
        out_ref[...] = jnp.zeros_like(out_ref)

    src = src_ref[...]
    dst = dst_ref[...]
    iota_s = jax.lax.broadcasted_iota(jnp.int32, (ke, n), 1)
    iota_d = jax.lax.broadcasted_iota(jnp.int32, (n, ke), 0)
    smask = (src == iota_s).astype(jnp.bfloat16)
    dmask = (dst == iota_d).astype(jnp.bfloat16)
    out_ref[...] += _dot(dmask, smask, 1, 0)


def _build_cnt_tc(edge_index, n):
    e = edge_index.shape[1]
    ke = 1024
    src = edge_index[0].reshape(e, 1)
    dst = edge_index[1].reshape(1, e)
    return pl.pallas_call(
        functools.partial(_cnt_body, n=n, ke=ke),
        grid=(e // ke,),
        in_specs=[
            pl.BlockSpec((ke, 1), lambda i: (i, 0)),
            pl.BlockSpec((1, ke), lambda i: (0, i)),
        ],
        out_specs=pl.BlockSpec((n, n), lambda i: (0, 0)),
        out_shape=jax.ShapeDtypeStruct((n, n), jnp.float32),
    )(src, dst)


_NW = 32
_ROWS = 32
_CH = 4096


def _build_cnt_sc(edge_index, n):
    e = edge_index.shape[1]
    phases = n // (_NW * _ROWS)
    n_chunks = e // _CH
    slab = _ROWS * n
    mesh = plsc.VectorSubcoreMesh(core_axis_name="c", subcore_axis_name="s")

    def body(src_hbm, dst_hbm, out_hbm, acc, sbuf, dbuf):
        wid = lax.axis_index("s") * 2 + lax.axis_index("c")
        ones = jnp.ones((16,), jnp.float32)
        zeros = jnp.zeros((16,), jnp.float32)
        for p in range(phases):
            lo = (p * _NW + wid) * _ROWS
            hi = lo + _ROWS

            def zero_body(i, c):
                acc[pl.ds(i * 16, 16)] = zeros
                return c

            lax.fori_loop(0, slab // 16, zero_body, 0)
            for ch in range(n_chunks):
                pltpu.sync_copy(src_hbm.at[pl.ds(ch * _CH, _CH)], sbuf)
                pltpu.sync_copy(dst_hbm.at[pl.ds(ch * _CH, _CH)], dbuf)

                def scat_body(i, c):
                    d = dbuf[pl.ds(i * 16, 16)]
                    s = sbuf[pl.ds(i * 16, 16)]
                    m = (d >= lo) & (d < hi)
                    idx = jnp.where(m, (d - lo) * n + s, 0)
                    plsc.addupdate_scatter(acc, [idx], ones, mask=m)
                    return c

                lax.fori_loop(0, _CH // 16, scat_body, 0)
            pltpu.sync_copy(acc, out_hbm.at[pl.ds(lo * n, slab)])

    flat = pl.kernel(
        body,
        mesh=mesh,
        compiler_params=pltpu.CompilerParams(needs_layout_passes=False),
        out_type=jax.ShapeDtypeStruct((n * n,), jnp.float32),
        scratch_types=[
            pltpu.VMEM((slab,), jnp.float32),
            pltpu.VMEM((_CH,), jnp.int32),
            pltpu.VMEM((_CH,), jnp.int32),
        ],
    )(edge_index[0], edge_index[1])
    return flat.reshape(n, n)


_build_cnt = _build_cnt_sc


def _prep_body(cnt_ref, anorm_ref, agcn_ref, *, n, br):
    i = pl.program_id(0)
    cnt = cnt_ref[...]
    deg = jnp.sum(cnt, axis=1, keepdims=True)
    anorm_ref[...] = cnt / jnp.maximum(deg, 1.0)
    rows = jax.lax.broadcasted_iota(jnp.int32, (br, n), 0) + i * br
    cols = jax.lax.broadcasted_iota(jnp.int32, (br, n), 1)
    cnti = cnt + jnp.where(rows == cols, 1.0, 0.0)
    agcn_ref[...] = cnti


def _prep(cnt, n):
    br = 256
    anorm, cnti = pl.pallas_call(
        functools.partial(_prep_body, n=n, br=br),
        grid=(n // br,),
        in_specs=[pl.BlockSpec((br, n), lambda i: (i, 0))],
        out_specs=[
            pl.BlockSpec((br, n), lambda i: (i, 0)),
            pl.BlockSpec((br, n), lambda i: (i, 0)),
        ],
        out_shape=[
            jax.ShapeDtypeStruct((n, n), jnp.float32),
            jax.ShapeDtypeStruct((n, n), jnp.float32),
        ],
    )(cnt)

    def _gcn_norm_body(cnti_ref, dinv_ref, dcol_ref, out_ref):
        out_ref[...] = cnti_ref[...] * dinv_ref[...] * dcol_ref[...]

    def _dinv_body(cnti_ref, out_ref):
        deg = jnp.sum(cnti_ref[...], axis=1, keepdims=True)
        out_ref[...] = jnp.where(
            deg > 0, jax.lax.rsqrt(jnp.maximum(deg, 1e-12)), 0.0
        )

    dinv = pl.pallas_call(
        _dinv_body,
        grid=(n // br,),
        in_specs=[pl.BlockSpec((br, n), lambda i: (i, 0))],
        out_specs=pl.BlockSpec((br, 1), lambda i: (i, 0)),
        out_shape=jax.ShapeDtypeStruct((n, 1), jnp.float32),
    )(cnti)
    agcn = pl.pallas_call(
        _gcn_norm_body,
        grid=(n // br,),
        in_specs=[
            pl.BlockSpec((br, n), lambda i: (i, 0)),
            pl.BlockSpec((1, n), lambda i: (0, 0)),
            pl.BlockSpec((br, 1), lambda i: (i, 0)),
        ],
        out_specs=pl.BlockSpec((br, n), lambda i: (i, 0)),
        out_shape=jax.ShapeDtypeStruct((n, n), jnp.float32),
    )(cnti, dinv.reshape(1, n), dinv)
    return anorm, agcn


def _colsage_meanT_body(e_ref, anorm_ref, out_ref):
    out_ref[...] = _dot(e_ref[...], anorm_ref[...], 1, 1)


def _colsage_mm_body(wl_ref, wr_ref, b_ref, meanT_ref, e_ref, out_ref):
    z = _dot(wl_ref[...], meanT_ref[...], 0, 0)
    z += _dot(wr_ref[...], e_ref[...], 0, 0)
    z += b_ref[...].reshape(-1, 1)
    out_ref[...] = _leaky(z)


def _rowsage_mean_body(anorm_ref, e_ref, out_ref):
    out_ref[...] = _dot(anorm_ref[...], e_ref[...], 1, 0)


def _rowsage_mm_body(mean_ref, e_ref, wl_ref, wr_ref, b_ref, out_ref):
    z = _dot(mean_ref[...], wl_ref[...], 1, 0)
    z += _dot(e_ref[...], wr_ref[...], 1, 0)
    z += b_ref[...]
    out_ref[...] = _leaky(z)


def _gcn_row_body(e_ref, gw_ref, gb_ref, agcn_ref, out_ref):
    h = _dot(e_ref[...], gw_ref[...], 1, 0)
    out_ref[...] = _leaky(_dot(agcn_ref[...], h, 1, 0) + gb_ref[...])


def _gcn_col_body(e_ref, gw_ref, gb_ref, agcn_ref, out_ref):
    h = _dot(e_ref[...], gw_ref[...], 0, 0)
    out_ref[...] = _leaky(_dot(agcn_ref[...], h, 1, 0) + gb_ref[...])


def _tconv_softmax_body(h_ref, cnt_ref, wq_ref, bq_ref, wk_ref, bk_ref,
                        wv_ref, bv_ref, ws_ref, bs_ref, out_ref, *, cd):
    h = h_ref[...]
    q = _dot(h, wq_ref[...], 1, 0) + bq_ref[...]
    k = _dot(h, wk_ref[...], 1, 0) + bk_ref[...]
    v = _dot(h, wv_ref[...], 1, 0) + bv_ref[...]
    s = _dot(q, k, 1, 1) * (1.0 / math.sqrt(cd))
    cnt = cnt_ref[...]
    mask = cnt > 0
    smax = jnp.max(jnp.where(mask, s, -1e30), axis=1, keepdims=True)
    smax = jnp.where(smax > -1e29, smax, 0.0)
    ex = jnp.where(mask, jnp.exp(s - smax), 0.0)
    den = jnp.sum(cnt * ex, axis=1, keepdims=True)
    p = cnt * ex / (den + 1e-16)
    out_ref[...] = _dot(p, v, 1, 0) + _dot(h, ws_ref[...], 1, 0) + bs_ref[...]


def _tconv_sigmoid_body(h_ref, cnt_ref, wq_ref, bq_ref, wk_ref, bk_ref,
                        wv_ref, bv_ref, ws_ref, bs_ref, out_ref,
                        *, cd, ecount, scale):
    h = h_ref[...]
    q = _dot(h, wq_ref[...], 1, 0) + bq_ref[...]
    k = _dot(h, wk_ref[...], 1, 0) + bk_ref[...]
    v = _dot(h, wv_ref[...], 1, 0) + bv_ref[...]
    s = _dot(q, k, 1, 1) * (1.0 / math.sqrt(cd))
    cnt = cnt_ref[...]
    m = jnp.sum(cnt * s) / ecount
    var = jnp.sum(cnt * (s - m) ** 2) / (ecount - 1.0)
    z = (s - m) * (scale * jax.lax.rsqrt(var))
    sig = 1.0 / (1.0 + jnp.exp(-z))
    out_ref[...] = (
        _dot(cnt * sig, v, 1, 0) + _dot(h, ws_ref[...], 1, 0) + bs_ref[...]
    )


def _decoder_body(c_ref, w1_ref, b1_ref, w2_ref, b2_ref, w3_ref, b3_ref,
                  out_ref):
    h = jnp.maximum(_dot(c_ref[...], w1_ref[...], 1, 0) + b1_ref[...], 0.0)
    h = jnp.maximum(_dot(h, w2_ref[...], 1, 0) + b2_ref[...], 0.0)
    out_ref[...] = _dot(h, w3_ref[...], 1, 0) + b3_ref[...]


def _call(body, out_shape, *args, **static):
    return pl.pallas_call(
        functools.partial(body, **static) if static else body,
        out_shape=jax.ShapeDtypeStruct(out_shape, jnp.float32),
    )(*args)


def _rowsage_mm(mean, e, wl, wr, b):
    bn = 512
    return pl.pallas_call(
        _rowsage_mm_body,
        grid=(_C // bn,),
        in_specs=[
            pl.BlockSpec((_G, _C), lambda j: (0, 0)),
            pl.BlockSpec((_G, _C), lambda j: (0, 0)),
            pl.BlockSpec((_C, bn), lambda j: (0, j)),
            pl.BlockSpec((_C, bn), lambda j: (0, j)),
            pl.BlockSpec((1, bn), lambda j: (0, j)),
        ],
        out_specs=pl.BlockSpec((_G, bn), lambda j: (0, j)),
        out_shape=jax.ShapeDtypeStruct((_G, _C), jnp.float32),
    )(mean, e, wl, wr, b.reshape(1, _C))


def kernel(x, knn_edge_index, ppi_edge_index, cWl0, cWr0, cb0, cWl1, cWr1,
           cb1, rWl0, rWr0, rb0, rWl1, rWr1, rb1, re_gW, re_gb, re_Wq, re_bq,
           re_Wk, re_bk, re_Wv, re_bv, re_Ws, re_bs, ce_gW, ce_gb, ce_Wq,
           ce_bq, ce_Wk, ce_bk, ce_Wv, ce_bv, ce_Ws, ce_bs, dW1, db1, dW2,
           db2, dW3, db3):
    cnt_knn = _build_cnt(knn_edge_index, _C)
    cnt_ppi = _build_cnt(ppi_edge_index, _G)
    anorm_knn, agcn_knn = _prep(cnt_knn, _C)
    anorm_ppi, agcn_ppi = _prep(cnt_ppi, _G)

    e = x
    for cWl, cWr, cb, rWl, rWr, rb in (
        (cWl0, cWr0, cb0, rWl0, rWr0, rb0),
        (cWl1, cWr1, cb1, rWl1, rWr1, rb1),
    ):
        meanT = _call(_colsage_meanT_body, (_G, _C), e, anorm_knn)
        e = _call(_colsage_mm_body, (_G, _C), cWl, cWr, cb, meanT, e)
        mean = _call(_rowsage_mean_body, (_G, _C), anorm_ppi, e)
        e = _rowsage_mm(mean, e, rWl, rWr, rb)

    hr = _call(_gcn_row_body, (_G, _INTER), e, re_gW, re_gb.reshape(1, -1),
               agcn_ppi)
    rows_embd = _call(
        _tconv_softmax_body, (_G, _EMB), hr, cnt_ppi,
        re_Wq, re_bq.reshape(1, -1), re_Wk, re_bk.reshape(1, -1),
        re_Wv, re_bv.reshape(1, -1), re_Ws, re_bs.reshape(1, -1),
        cd=_EMB)

    hc = _call(_gcn_col_body, (_C, _INTER), e, ce_gW, ce_gb.reshape(1, -1),
               agcn_knn)
    cols_embd = _call(
        _tconv_sigmoid_body, (_C, _EMB), hc, cnt_knn,
        ce_Wq, ce_bq.reshape(1, -1), ce_Wk, ce_bk.reshape(1, -1),
        ce_Wv, ce_bv.reshape(1, -1), ce_Ws, ce_bs.reshape(1, -1),
        cd=_EMB, ecount=float(_EKNN), scale=3.0)

    out_features = _call(
        _decoder_body, (_C, _G), cols_embd, dW1, db1.reshape(1, -1),
        dW2, db2.reshape(1, -1), dW3, db3.reshape(1, -1))

    return (rows_embd, cols_embd, out_features)


# speedup vs baseline: 17.2784x; 1.1687x over previous
import functools
import math

import jax
import jax.numpy as jnp
from jax import lax
from jax.experimental import pallas as pl
from jax.experimental.pallas import tpu as pltpu
from jax.experimental.pallas import tpu_sc as plsc

_G, _C, _EPPI, _EKNN, _INTER, _EMB = 1024, 2048, 32768, 65536, 512, 128


def _leaky(v):
    return jnp.where(v > 0, v, 0.01 * v)


def _rawdot(a, b, ca, cb):
    return jax.lax.dot_general(
        a, b, (((ca,), (cb,)), ((), ())), preferred_element_type=jnp.float32
    )


_dot = _rawdot


def _split(x):
    hi = x.astype(jnp.bfloat16)
    lo = (x - hi.astype(jnp.float32)).astype(jnp.bfloat16)
    return hi, lo


def _dot2(cntb, x, ca, cb):
    xh, xl = _split(x)
    return _rawdot(cntb, xh, ca, cb) + _rawdot(cntb, xl, ca, cb)


def _dot2r(x, cntb, ca, cb):
    xh, xl = _split(x)
    return _rawdot(xh, cntb, ca, cb) + _rawdot(xl, cntb, ca, cb)


def _dot3(a, b, ca, cb):
    ah, al = _split(a)
    bh, bl = _split(b)
    return (_rawdot(ah, bh, ca, cb) + _rawdot(ah, bl, ca, cb)
            + _rawdot(al, bh, ca, cb))


def _cnt_body(src_ref, dst_ref, out_ref, *, n, ke):
    i = pl.program_id(0)

    @pl.when(i == 0)
    def _():
        out_ref[...] = jnp.zeros_like(out_ref)

    src = src_ref[...]
    dst = dst_ref[...]
    iota_s = jax.lax.broadcasted_iota(jnp.int32, (ke, n), 1)
    iota_d = jax.lax.broadcasted_iota(jnp.int32, (n, ke), 0)
    smask = (src == iota_s).astype(jnp.bfloat16)
    dmask = (dst == iota_d).astype(jnp.bfloat16)
    out_ref[...] += _dot(dmask, smask, 1, 0)


def _build_cnt_tc(edge_index, n):
    e = edge_index.shape[1]
    ke = 1024
    src = edge_index[0].reshape(e, 1)
    dst = edge_index[1].reshape(1, e)
    return pl.pallas_call(
        functools.partial(_cnt_body, n=n, ke=ke),
        grid=(e // ke,),
        in_specs=[
            pl.BlockSpec((ke, 1), lambda i: (i, 0)),
            pl.BlockSpec((1, ke), lambda i: (0, i)),
        ],
        out_specs=pl.BlockSpec((n, n), lambda i: (0, 0)),
        out_shape=jax.ShapeDtypeStruct((n, n), jnp.float32),
    )(src, dst)


_NW = 32
_ROWS = 32
_CH = 8192


def _build_cnt_sc(edge_index, n):
    e = edge_index.shape[1]
    phases = n // (_NW * _ROWS)
    n_chunks = e // _CH
    slab = _ROWS * n
    mesh = plsc.VectorSubcoreMesh(core_axis_name="c", subcore_axis_name="s")

    def body(src_hbm, dst_hbm, out_hbm, acc, s0, d0, s1, d1, sem0, sem1):
        wid = lax.axis_index("s") * 2 + lax.axis_index("c")
        ones = jnp.ones((16,), jnp.float32)
        zeros = jnp.zeros((16,), jnp.float32)
        bufs = ((s0, d0, sem0), (s1, d1, sem1))

        def stage(ch, slot):
            s_ref, d_ref, sem = bufs[slot]
            return (
                pltpu.async_copy(src_hbm.at[pl.ds(ch * _CH, _CH)], s_ref, sem),
                pltpu.async_copy(dst_hbm.at[pl.ds(ch * _CH, _CH)], d_ref, sem),
            )

        pending = stage(0, 0)
        for p in range(phases):
            lo = (p * _NW + wid) * _ROWS
            hi = lo + _ROWS

            @plsc.parallel_loop(0, slab // 16, unroll=8)
            def _(i):
                acc[pl.ds(i * 16, 16)] = zeros

            for ch in range(n_chunks):
                s_ref, d_ref, _sem = bufs[ch % 2]
                for h in pending:
                    h.wait()
                nxt = ch + 1 if ch + 1 < n_chunks else (0 if p + 1 < phases
                                                        else -1)
                if nxt >= 0:
                    pending = stage(nxt, (ch + 1) % 2)
                else:
                    pending = ()

                def scat_body(i, c):
                    d = d_ref[pl.ds(i * 16, 16)]
                    s = s_ref[pl.ds(i * 16, 16)]
                    m = (d >= lo) & (d < hi)
                    idx = jnp.where(m, (d - lo) * n + s, 0)
                    plsc.addupdate_scatter(acc, [idx], ones, mask=m)
                    return c

                lax.fori_loop(0, _CH // 16, scat_body, 0)

            pltpu.sync_copy(acc, out_hbm.at[pl.ds(lo * n, slab)])

    flat = pl.kernel(
        body,
        mesh=mesh,
        compiler_params=pltpu.CompilerParams(needs_layout_passes=False),
        out_type=jax.ShapeDtypeStruct((n * n,), jnp.float32),
        scratch_types=[
            pltpu.VMEM((slab,), jnp.float32),
            pltpu.VMEM((_CH,), jnp.int32),
            pltpu.VMEM((_CH,), jnp.int32),
            pltpu.VMEM((_CH,), jnp.int32),
            pltpu.VMEM((_CH,), jnp.int32),
            pltpu.SemaphoreType.DMA,
            pltpu.SemaphoreType.DMA,
        ],
    )(edge_index[0], edge_index[1])
    return flat.reshape(n, n)


_build_cnt = _build_cnt_sc


def _prep_body(cnt_ref, cntb_ref, cntib_ref, rdeg_ref, dinv_ref, *, n, br):
    i = pl.program_id(0)
    cnt = cnt_ref[...]
    deg = jnp.sum(cnt, axis=1, keepdims=True)
    cntb_ref[...] = cnt.astype(jnp.bfloat16)
    rows = jax.lax.broadcasted_iota(jnp.int32, (br, n), 0) + i * br
    cols = jax.lax.broadcasted_iota(jnp.int32, (br, n), 1)
    cnti = cnt + jnp.where(rows == cols, 1.0, 0.0)
    cntib_ref[...] = cnti.astype(jnp.bfloat16)
    rdeg_ref[...] = 1.0 / jnp.maximum(deg, 1.0)
    dinv_ref[...] = jax.lax.rsqrt(deg + 1.0)


def _prep(cnt, n):
    br = 256
    return pl.pallas_call(
        functools.partial(_prep_body, n=n, br=br),
        grid=(n // br,),
        in_specs=[pl.BlockSpec((br, n), lambda i: (i, 0))],
        out_specs=[
            pl.BlockSpec((br, n), lambda i: (i, 0)),
            pl.BlockSpec((br, n), lambda i: (i, 0)),
            pl.BlockSpec((br, 1), lambda i: (i, 0)),
            pl.BlockSpec((br, 1), lambda i: (i, 0)),
        ],
        out_shape=[
            jax.ShapeDtypeStruct((n, n), jnp.bfloat16),
            jax.ShapeDtypeStruct((n, n), jnp.bfloat16),
            jax.ShapeDtypeStruct((n, 1), jnp.float32),
            jax.ShapeDtypeStruct((n, 1), jnp.float32),
        ],
    )(cnt)


def _colsage_meanT_body(e_ref, cntb_ref, rdeg_ref, out_ref):
    out_ref[...] = _dot2r(e_ref[...], cntb_ref[...], 1, 1) * rdeg_ref[...]


def _colsage_mm_body(wl_ref, wr_ref, b_ref, meanT_ref, e_ref, out_ref):
    z = _dot(wl_ref[...], meanT_ref[...], 0, 0)
    z += _dot(wr_ref[...], e_ref[...], 0, 0)
    z += b_ref[...].reshape(-1, 1)
    out_ref[...] = _leaky(z)


def _rowsage_mean_body(cntb_ref, e_ref, rdeg_ref, out_ref):
    out_ref[...] = _dot2(cntb_ref[...], e_ref[...], 1, 0) * rdeg_ref[...]


def _rowsage_mm_body(mean_ref, e_ref, wl_ref, wr_ref, b_ref, out_ref):
    z = _dot(mean_ref[...], wl_ref[...], 1, 0)
    z += _dot(e_ref[...], wr_ref[...], 1, 0)
    z += b_ref[...]
    out_ref[...] = _leaky(z)


def _gcn_row_body(e_ref, gw_ref, gb_ref, cntib_ref, dinv_ref, out_ref):
    h = _dot(e_ref[...], gw_ref[...], 1, 0)
    dinv = dinv_ref[...]
    t = _dot2(cntib_ref[...], h * dinv, 1, 0)
    out_ref[...] = _leaky(t * dinv + gb_ref[...])


def _gcn_col_body(e_ref, gw_ref, gb_ref, cntib_ref, dinv_ref, out_ref):
    h = _dot(e_ref[...], gw_ref[...], 0, 0)
    dinv = dinv_ref[...]
    t = _dot2(cntib_ref[...], h * dinv, 1, 0)
    out_ref[...] = _leaky(t * dinv + gb_ref[...])


def _tconv_softmax_body(h_ref, cnt_ref, wq_ref, bq_ref, wk_ref, bk_ref,
                        wv_ref, bv_ref, ws_ref, bs_ref, out_ref, *, cd):
    h = h_ref[...]
    q = _dot(h, wq_ref[...], 1, 0) + bq_ref[...]
    k = _dot(h, wk_ref[...], 1, 0) + bk_ref[...]
    v = _dot(h, wv_ref[...], 1, 0) + bv_ref[...]
    s = _dot3(q, k, 1, 1) * (1.0 / math.sqrt(cd))
    cnt = cnt_ref[...].astype(jnp.float32)
    mask = cnt > 0
    smax = jnp.max(jnp.where(mask, s, -1e30), axis=1, keepdims=True)
    smax = jnp.where(smax > -1e29, smax, 0.0)
    ex = jnp.where(mask, jnp.exp(s - smax), 0.0)
    den = jnp.sum(cnt * ex, axis=1, keepdims=True)
    p = cnt * ex / (den + 1e-16)
    out_ref[...] = (_dot3(p, v, 1, 0) + _dot(h, ws_ref[...], 1, 0)
                    + bs_ref[...])


def _qkvs_body(h_ref, wq_ref, bq_ref, wk_ref, bk_ref, wv_ref, bv_ref,
               ws_ref, bs_ref, q_ref, k_ref, v_ref, hs_ref):
    h = h_ref[...]
    q_ref[...] = _dot(h, wq_ref[...], 1, 0) + bq_ref[...]
    k_ref[...] = _dot(h, wk_ref[...], 1, 0) + bk_ref[...]
    v_ref[...] = _dot(h, wv_ref[...], 1, 0) + bv_ref[...]
    hs_ref[...] = _dot(h, ws_ref[...], 1, 0) + bs_ref[...]


def _sig_stats_body(q_ref, k_ref, cnt_ref, out_ref, *, cd):
    s = _dot3(q_ref[...], k_ref[...], 1, 1) * (1.0 / math.sqrt(cd))
    cnt = cnt_ref[...].astype(jnp.float32)
    s1 = jnp.sum(cnt * s)
    s2 = jnp.sum(cnt * s * s)
    lane = jax.lax.broadcasted_iota(jnp.int32, (1, 1, 128), 2)
    out_ref[...] = jnp.where(lane == 0, s1, jnp.where(lane == 1, s2, 0.0))


def _sig_apply_body(q_ref, k_ref, v_ref, hs_ref, cnt_ref, part_ref, out_ref,
                    *, cd, ecount, scale):
    part = part_ref[...]
    lane = jax.lax.broadcasted_iota(jnp.int32, part.shape, 2)
    s1 = jnp.sum(jnp.where(lane == 0, part, 0.0))
    s2 = jnp.sum(jnp.where(lane == 1, part, 0.0))
    m = s1 / ecount
    var = (s2 - ecount * m * m) / (ecount - 1.0)
    s = _dot3(q_ref[...], k_ref[...], 1, 1) * (1.0 / math.sqrt(cd))
    z = (s - m) * (scale * jax.lax.rsqrt(var))
    sig = 1.0 / (1.0 + jnp.exp(-z))
    p = cnt_ref[...].astype(jnp.float32) * sig
    out_ref[...] = _dot3(p, v_ref[...], 1, 0) + hs_ref[...]


def _tconv_sigmoid(h, cnt, wq, bq, wk, bk, wv, bv, ws, bs, n, cd, ecount,
                   scale):
    q, k, v, hs = pl.pallas_call(
        _qkvs_body,
        out_shape=[jax.ShapeDtypeStruct((n, cd), jnp.float32)] * 4,
    )(h, wq, bq.reshape(1, -1), wk, bk.reshape(1, -1), wv, bv.reshape(1, -1),
      ws, bs.reshape(1, -1))
    nb, br = 4, n // 4
    part = pl.pallas_call(
        functools.partial(_sig_stats_body, cd=cd),
        grid=(nb,),
        in_specs=[
            pl.BlockSpec((br, cd), lambda i: (i, 0)),
            pl.BlockSpec((n, cd), lambda i: (0, 0)),
            pl.BlockSpec((br, n), lambda i: (i, 0)),
        ],
        out_specs=pl.BlockSpec((1, 1, 128), lambda i: (i, 0, 0)),
        out_shape=jax.ShapeDtypeStruct((nb, 1, 128), jnp.float32),
    )(q, k, cnt)
    return pl.pallas_call(
        functools.partial(_sig_apply_body, cd=cd, ecount=ecount, scale=scale),
        grid=(nb,),
        in_specs=[
            pl.BlockSpec((br, cd), lambda i: (i, 0)),
            pl.BlockSpec((n, cd), lambda i: (0, 0)),
            pl.BlockSpec((n, cd), lambda i: (0, 0)),
            pl.BlockSpec((br, cd), lambda i: (i, 0)),
            pl.BlockSpec((br, n), lambda i: (i, 0)),
            pl.BlockSpec((nb, 1, 128), lambda i: (0, 0, 0)),
        ],
        out_specs=pl.BlockSpec((br, cd), lambda i: (i, 0)),
        out_shape=jax.ShapeDtypeStruct((n, cd), jnp.float32),
    )(q, k, v, hs, cnt, part)


def _decoder_body(c_ref, w1_ref, b1_ref, w2_ref, b2_ref, w3_ref, b3_ref,
                  out_ref):
    h = jnp.maximum(_dot(c_ref[...], w1_ref[...], 1, 0) + b1_ref[...], 0.0)
    h = jnp.maximum(_dot(h, w2_ref[...], 1, 0) + b2_ref[...], 0.0)
    out_ref[...] = _dot(h, w3_ref[...], 1, 0) + b3_ref[...]


def _call(body, out_shape, *args, **static):
    return pl.pallas_call(
        functools.partial(body, **static) if static else body,
        out_shape=jax.ShapeDtypeStruct(out_shape, jnp.float32),
    )(*args)


def _rowsage_mm(mean, e, wl, wr, b):
    bn = 512
    return pl.pallas_call(
        _rowsage_mm_body,
        grid=(_C // bn,),
        in_specs=[
            pl.BlockSpec((_G, _C), lambda j: (0, 0)),
            pl.BlockSpec((_G, _C), lambda j: (0, 0)),
            pl.BlockSpec((_C, bn), lambda j: (0, j)),
            pl.BlockSpec((_C, bn), lambda j: (0, j)),
            pl.BlockSpec((1, bn), lambda j: (0, j)),
        ],
        out_specs=pl.BlockSpec((_G, bn), lambda j: (0, j)),
        out_shape=jax.ShapeDtypeStruct((_G, _C), jnp.float32),
    )(mean, e, wl, wr, b.reshape(1, _C))


def kernel(x, knn_edge_index, ppi_edge_index, cWl0, cWr0, cb0, cWl1, cWr1,
           cb1, rWl0, rWr0, rb0, rWl1, rWr1, rb1, re_gW, re_gb, re_Wq, re_bq,
           re_Wk, re_bk, re_Wv, re_bv, re_Ws, re_bs, ce_gW, ce_gb, ce_Wq,
           ce_bq, ce_Wk, ce_bk, ce_Wv, ce_bv, ce_Ws, ce_bs, dW1, db1, dW2,
           db2, dW3, db3):
    cnt_knn = _build_cnt(knn_edge_index, _C)
    cnt_ppi = _build_cnt(ppi_edge_index, _G)
    cntb_knn, cntib_knn, rdeg_knn, dinv_knn = _prep(cnt_knn, _C)
    cntb_ppi, cntib_ppi, rdeg_ppi, dinv_ppi = _prep(cnt_ppi, _G)

    e = x
    for cWl, cWr, cb, rWl, rWr, rb in (
        (cWl0, cWr0, cb0, rWl0, rWr0, rb0),
        (cWl1, cWr1, cb1, rWl1, rWr1, rb1),
    ):
        meanT = _call(_colsage_meanT_body, (_G, _C), e, cntb_knn,
                      rdeg_knn.reshape(1, _C))
        e = _call(_colsage_mm_body, (_G, _C), cWl, cWr, cb, meanT, e)
        mean = _call(_rowsage_mean_body, (_G, _C), cntb_ppi, e, rdeg_ppi)
        e = _rowsage_mm(mean, e, rWl, rWr, rb)

    hr = _call(_gcn_row_body, (_G, _INTER), e, re_gW, re_gb.reshape(1, -1),
               cntib_ppi, dinv_ppi)
    rows_embd = _call(
        _tconv_softmax_body, (_G, _EMB), hr, cntb_ppi,
        re_Wq, re_bq.reshape(1, -1), re_Wk, re_bk.reshape(1, -1),
        re_Wv, re_bv.reshape(1, -1), re_Ws, re_bs.reshape(1, -1),
        cd=_EMB)

    hc = _call(_gcn_col_body, (_C, _INTER), e, ce_gW, ce_gb.reshape(1, -1),
               cntib_knn, dinv_knn)
    cols_embd = _tconv_sigmoid(
        hc, cntb_knn, ce_Wq, ce_bq, ce_Wk, ce_bk, ce_Wv, ce_bv, ce_Ws, ce_bs,
        _C, _EMB, float(_EKNN), 3.0)

    out_features = _call(
        _decoder_body, (_C, _G), cols_embd, dW1, db1.reshape(1, -1),
        dW2, db2.reshape(1, -1), dW3, db3.reshape(1, -1))

    return (rows_embd, cols_embd, out_features)
